# Initial kernel scaffold; baseline (speedup 1.0000x reference)
import jax
import jax.numpy as jnp
from jax.experimental import pallas as pl


def kernel(indices, emb_table, fc_w, fc_b):
    raise NotImplementedError("write your pallas kernel here")


# speedup vs baseline: 2.5162x; 2.5162x over previous
import functools

import jax
import jax.numpy as jnp
from jax import lax
from jax.experimental import pallas as pl
from jax.experimental.pallas import tpu as pltpu
from jax.experimental.pallas import tpu_sc as plsc

BATCH = 16384
SEQ = 50
DIM = 64
NCLS = 2

LANES = 16
NVEC = DIM // LANES

CH = 8
TOK = CH * SEQ


@functools.cache
def _build():
  info = plsc.get_sparse_core_info()
  nw = info.num_cores * info.num_subcores
  sent_w = BATCH // nw
  tok_w = sent_w * SEQ
  nch = sent_w // CH

  mesh = plsc.VectorSubcoreMesh(core_axis_name="c", subcore_axis_name="s")

  @functools.partial(
      pl.kernel,
      out_type=jax.ShapeDtypeStruct((BATCH * NCLS,), jnp.float32),
      mesh=mesh,
      compiler_params=pltpu.CompilerParams(
          needs_layout_passes=False, use_tc_tiling_on_sc=False),
      scratch_types=[
          pltpu.VMEM((tok_w,), jnp.int32),
          pltpu.VMEM((TOK, DIM), jnp.float32),
          pltpu.VMEM((NCLS, DIM), jnp.float32),
          pltpu.VMEM((LANES,), jnp.float32),
          pltpu.VMEM((sent_w * NCLS,), jnp.float32),
          pltpu.SemaphoreType.DMA,
      ],
  )
  def fasttext_kernel(idx_hbm, table_hbm, fcw_hbm, fcb_hbm, out_hbm,
                      idx_v, rows_v, w_v, b_v, out_v, sem):
    wid = lax.axis_index("s") * info.num_cores + lax.axis_index("c")

    pltpu.sync_copy(idx_hbm.at[pl.ds(wid * tok_w, tok_w)], idx_v)
    pltpu.sync_copy(fcw_hbm, w_v)
    pltpu.sync_copy(fcb_hbm, b_v)

    w = [[w_v[c, pl.ds(LANES * j, LANES)] for j in range(NVEC)]
         for c in range(NCLS)]
    bias_vec = b_v[pl.ds(0, LANES)]
    lane = lax.iota(jnp.int32, LANES)
    inv_seq = jnp.float32(1.0 / SEQ)

    @pl.loop(0, nch)
    def _chunk(g):
      pltpu.async_copy(
          table_hbm.at[idx_v.at[pl.ds(g * TOK, TOK)]], rows_v, sem
      ).wait()
      outvec = jnp.zeros((LANES,), jnp.float32)
      for s in range(CH):
        base = s * SEQ
        acc = tuple(rows_v[base, pl.ds(LANES * j, LANES)] for j in range(NVEC))

        def tok_body(t, carry, base=base):
          return tuple(
              carry[j] + rows_v[base + t, pl.ds(LANES * j, LANES)]
              for j in range(NVEC)
          )

        acc = lax.fori_loop(1, SEQ, tok_body, acc, unroll=7)
        for c in range(NCLS):
          prod = acc[0] * w[c][0]
          for j in range(1, NVEC):
            prod = prod + acc[j] * w[c][j]
          outvec = jnp.where(lane == (s * NCLS + c), jnp.sum(prod), outvec)
      out_v[pl.ds(g * LANES, LANES)] = outvec * inv_seq + bias_vec

    pltpu.sync_copy(out_v, out_hbm.at[pl.ds(wid * sent_w * NCLS,
                                            sent_w * NCLS)])

  return fasttext_kernel


def kernel(indices, emb_table, fc_w, fc_b):
  idx_flat = indices.reshape(-1).astype(jnp.int32)
  fcb_tiled = jnp.tile(fc_b.astype(jnp.float32), LANES // NCLS)
  out_flat = _build()(idx_flat, emb_table, fc_w, fcb_tiled)
  return out_flat.reshape(BATCH, NCLS)
